# Initial kernel scaffold; baseline (speedup 1.0000x reference)
import jax
import jax.numpy as jnp
from jax.experimental import pallas as pl


def kernel(t, x_int_t, boundary_values, edge_index_int, edge_index_bound, timestamps, Wm1, bm1, Ws1, bs1, Wu1, bu1, Wm2, bm2, Ws2, bs2, Wu2, bu2, Wm3, bm3, Ws3, bs3, Wu3, bu3, Wm4, bm4, Ws4, bs4, Wu4, bu4):
    raise NotImplementedError("write your pallas kernel here")


# speedup vs baseline: 13.7669x; 13.7669x over previous
import functools

import jax
import jax.numpy as jnp
from jax import lax
from jax.experimental import pallas as pl
from jax.experimental.pallas import tpu as pltpu
from jax.experimental.pallas import tpu_sc as plsc

_N = 50000
_E = 800000
_MSG = 16
_NC = 2
_NS = 16
_NW = _NC * _NS
_CHUNK = 1000
_E_PER_TILE = _E // _NW
_N_CHUNKS = _E_PER_TILE // _CHUNK
_RPT = 3128
_RPT_LAST = _N - 15 * _RPT

_f32 = jnp.float32


def _make_edge_agg():
  mesh = plsc.VectorSubcoreMesh(core_axis_name="c", subcore_axis_name="s")
  out_type = [jax.ShapeDtypeStruct((_NC, _N, _MSG), _f32)]
  scratch = [
      pltpu.VMEM((_CHUNK,), jnp.int32),
      pltpu.VMEM((_CHUNK,), jnp.int32),
      pltpu.VMEM((_CHUNK, _MSG), _f32),
      pltpu.VMEM((_RPT, _MSG), _f32),
      pltpu.VMEM_SHARED((_N, _MSG), _f32),
      pltpu.SemaphoreType.DMA,
  ]

  @functools.partial(
      pl.kernel, mesh=mesh, out_type=out_type, scratch_types=scratch,
      compiler_params=pltpu.CompilerParams(use_tc_tiling_on_sc=False))
  def edge_agg(a_hbm, src_hbm, dst_hbm, s_out, idx_s, idx_d, rows, zbuf,
               s_sh, sem):
    cid = lax.axis_index("c")
    sid = lax.axis_index("s")
    wid = cid * _NS + sid

    def zrow(i, carry):
      zbuf[i, :] = jnp.zeros((_MSG,), _f32)
      return carry
    lax.fori_loop(0, _RPT, zrow, 0)
    acc0 = sid * _RPT
    last = sid == _NS - 1

    def _copy_rows(src_at, dst_at):
      @pl.when(jnp.logical_not(last))
      def _():
        pltpu.sync_copy(src_at(_RPT), dst_at(_RPT))

      @pl.when(last)
      def _():
        pltpu.sync_copy(src_at(_RPT_LAST), dst_at(_RPT_LAST))

    _copy_rows(lambda n: zbuf.at[pl.ds(0, n)],
               lambda n: s_sh.at[pl.ds(acc0, n)])
    plsc.subcore_barrier()

    base = wid * _E_PER_TILE

    def chunk(k, carry):
      e0 = base + k * _CHUNK
      pltpu.sync_copy(src_hbm.at[pl.ds(e0, _CHUNK)], idx_s)
      pltpu.sync_copy(dst_hbm.at[pl.ds(e0, _CHUNK)], idx_d)
      pltpu.async_copy(a_hbm.at[idx_s], rows, sem).wait()
      pltpu.sync_copy(rows, s_sh.at[idx_d], add=True)
      return carry
    lax.fori_loop(0, _N_CHUNKS, chunk, 0)
    plsc.subcore_barrier()

    _copy_rows(lambda n: s_sh.at[pl.ds(acc0, n)],
               lambda n: s_out.at[cid, pl.ds(acc0, n)])

  return edge_agg


def _make_edge_count():
  mesh = plsc.VectorSubcoreMesh(core_axis_name="c", subcore_axis_name="s")
  out_type = [jax.ShapeDtypeStruct((_NC, _N, _MSG), _f32)]
  scratch = [
      pltpu.VMEM((_CHUNK,), jnp.int32),
      pltpu.VMEM((_CHUNK, _MSG), _f32),
      pltpu.VMEM((_RPT, _MSG), _f32),
      pltpu.VMEM_SHARED((_N, _MSG), _f32),
  ]

  @functools.partial(
      pl.kernel, mesh=mesh, out_type=out_type, scratch_types=scratch,
      compiler_params=pltpu.CompilerParams(use_tc_tiling_on_sc=False))
  def edge_count(dst_hbm, c_out, idx_d, ones, zbuf, c_sh):
    cid = lax.axis_index("c")
    sid = lax.axis_index("s")
    wid = cid * _NS + sid

    def zrow(i, carry):
      zbuf[i, :] = jnp.zeros((_MSG,), _f32)
      return carry
    lax.fori_loop(0, _RPT, zrow, 0)

    def orow(i, carry):
      ones[i, :] = jnp.ones((_MSG,), _f32)
      return carry
    lax.fori_loop(0, _CHUNK, orow, 0)
    acc0 = sid * _RPT
    last = sid == _NS - 1

    def _copy_rows(src_at, dst_at):
      @pl.when(jnp.logical_not(last))
      def _():
        pltpu.sync_copy(src_at(_RPT), dst_at(_RPT))

      @pl.when(last)
      def _():
        pltpu.sync_copy(src_at(_RPT_LAST), dst_at(_RPT_LAST))

    _copy_rows(lambda n: zbuf.at[pl.ds(0, n)],
               lambda n: c_sh.at[pl.ds(acc0, n)])
    plsc.subcore_barrier()

    base = wid * _E_PER_TILE

    def chunk(k, carry):
      e0 = base + k * _CHUNK
      pltpu.sync_copy(dst_hbm.at[pl.ds(e0, _CHUNK)], idx_d)
      pltpu.sync_copy(ones, c_sh.at[idx_d], add=True)
      return carry
    lax.fori_loop(0, _N_CHUNKS, chunk, 0)
    plsc.subcore_barrier()

    _copy_rows(lambda n: c_sh.at[pl.ds(acc0, n)],
               lambda n: c_out.at[cid, pl.ds(acc0, n)])

  return edge_count


_edge_agg = _make_edge_agg()
_edge_count = _make_edge_count()


_BN = 2000
_GRID = _N // _BN


def _softplus(x):
  return jnp.maximum(x, 0.0) + jnp.log1p(jnp.exp(-jnp.abs(x)))


def _dot(a, b):
  return jnp.dot(a, b, preferred_element_type=_f32,
                 precision=lax.Precision.HIGHEST)


def _full(shape):
  return pl.BlockSpec(shape, lambda i: (0, 0))


def _rows(width):
  return pl.BlockSpec((_BN, width), lambda i: (i, 0))


def _tc_pre(x, wms_t, wmd_t, ws_t, bm, bs):
  din = x.shape[1]

  def body(x_r, wms_r, wmd_r, ws_r, bm_r, bs_r, a_r, bp_r, xs_r):
    xv = x_r[...]
    a_r[...] = _dot(xv, wms_r[...])
    bp_r[...] = _dot(xv, wmd_r[...]) + bm_r[...]
    xs_r[...] = _dot(xv, ws_r[...]) + bs_r[...]

  return pl.pallas_call(
      body,
      grid=(_GRID,),
      in_specs=[_rows(din), _full(wms_t.shape), _full(wmd_t.shape),
                _full(ws_t.shape), _full(bm.shape), _full(bs.shape)],
      out_specs=[_rows(_MSG), _rows(_MSG), _rows(ws_t.shape[1])],
      out_shape=[jax.ShapeDtypeStruct((_N, _MSG), _f32),
                 jax.ShapeDtypeStruct((_N, _MSG), _f32),
                 jax.ShapeDtypeStruct((_N, ws_t.shape[1]), _f32)],
  )(x, wms_t, wmd_t, ws_t, bm, bs)


def _agg_from_parts(s0, s1, c0, c1, bp):
  cnt = c0 + c1
  ssum = s0 + s1
  return jnp.where(cnt > 0.5, ssum / jnp.maximum(cnt, 1.0) + bp, 0.0)


def _tc_mid(sp, cp, bp, xs, wu_t, bu, wms_t, wmd_t, ws_t, bm_n, bs_n):
  dnext = ws_t.shape[1]

  def body(s0_r, s1_r, c0_r, c1_r, bp_r, xs_r, wu_r, bu_r, wms_r, wmd_r,
           ws_r, bmn_r, bsn_r, a_r, bpn_r, xsn_r):
    agg = _agg_from_parts(s0_r[...], s1_r[...], c0_r[...], c1_r[...],
                          bp_r[...])
    h = _softplus(xs_r[...] + _dot(agg, wu_r[...]) + bu_r[...])
    a_r[...] = _dot(h, wms_r[...])
    bpn_r[...] = _dot(h, wmd_r[...]) + bmn_r[...]
    xsn_r[...] = _dot(h, ws_r[...]) + bsn_r[...]

  return pl.pallas_call(
      body,
      grid=(_GRID,),
      in_specs=[_rows(_MSG)] * 4 + [_rows(_MSG), _rows(xs.shape[1]),
                _full(wu_t.shape), _full(bu.shape), _full(wms_t.shape),
                _full(wmd_t.shape), _full(ws_t.shape), _full(bm_n.shape),
                _full(bs_n.shape)],
      out_specs=[_rows(_MSG), _rows(_MSG), _rows(dnext)],
      out_shape=[jax.ShapeDtypeStruct((_N, _MSG), _f32),
                 jax.ShapeDtypeStruct((_N, _MSG), _f32),
                 jax.ShapeDtypeStruct((_N, dnext), _f32)],
  )(sp[0], sp[1], cp[0], cp[1], bp, xs, wu_t, bu, wms_t, wmd_t, ws_t,
    bm_n, bs_n)


def _tc_final(sp, cp, bp, xs, wu_t, bu):
  dout = wu_t.shape[1]

  def body(s0_r, s1_r, c0_r, c1_r, bp_r, xs_r, wu_r, bu_r, o_r):
    agg = _agg_from_parts(s0_r[...], s1_r[...], c0_r[...], c1_r[...],
                          bp_r[...])
    o_r[...] = xs_r[...] + _dot(agg, wu_r[...]) + bu_r[...]

  return pl.pallas_call(
      body,
      grid=(_GRID,),
      in_specs=[_rows(_MSG)] * 4 + [_rows(_MSG), _rows(xs.shape[1]),
                _full(wu_t.shape), _full(bu.shape)],
      out_specs=_rows(dout),
      out_shape=jax.ShapeDtypeStruct((_N, dout), _f32),
  )(sp[0], sp[1], cp[0], cp[1], bp, xs, wu_t, bu)


def kernel(t, x_int_t, boundary_values, edge_index_int, edge_index_bound,
           timestamps, Wm1, bm1, Ws1, bs1, Wu1, bu1, Wm2, bm2, Ws2, bs2,
           Wu2, bu2, Wm3, bm3, Ws3, bs3, Wu3, bu3, Wm4, bm4, Ws4, bs4,
           Wu4, bu4):
  x = x_int_t[0]
  src = edge_index_int[0]
  dst = edge_index_int[1]

  layers = [(Wm1, bm1, Ws1, bs1, Wu1, bu1), (Wm2, bm2, Ws2, bs2, Wu2, bu2),
            (Wm3, bm3, Ws3, bs3, Wu3, bu3), (Wm4, bm4, Ws4, bs4, Wu4, bu4)]

  def split(l):
    Wm, bm, Ws, bs, Wu, bu = layers[l]
    din = Wm.shape[1] // 2
    return (Wm[:, :din].T, Wm[:, din:].T, Ws.T, Wu.T,
            bm.reshape(1, -1), bs.reshape(1, -1), bu.reshape(1, -1))

  wms, wmd, wst, wut, bm_, bs_, bu_ = zip(*[split(l) for l in range(4)])

  cp = _edge_count(dst)[0]
  a, bp, xs = _tc_pre(x, wms[0], wmd[0], wst[0], bm_[0], bs_[0])
  sp = _edge_agg(a, src, dst)[0]
  for l in (1, 2, 3):
    a, bp, xs = _tc_mid(sp, cp, bp, xs, wut[l - 1], bu_[l - 1], wms[l],
                        wmd[l], wst[l], bm_[l], bs_[l])
    sp = _edge_agg(a, src, dst)[0]
  x_out = _tc_final(sp, cp, bp, xs, wut[3], bu_[3])

  return (x_out[None], jnp.zeros_like(boundary_values),
          jnp.zeros_like(edge_index_int), jnp.zeros_like(edge_index_bound),
          jnp.zeros_like(timestamps))


# speedup vs baseline: 16.8020x; 1.2205x over previous
import functools

import jax
import jax.numpy as jnp
from jax import lax
from jax.experimental import pallas as pl
from jax.experimental.pallas import tpu as pltpu
from jax.experimental.pallas import tpu_sc as plsc

_N = 50000
_E = 800000
_MSG = 16
_NC = 2
_NS = 16
_NW = _NC * _NS
_CHUNK = 1000
_E_PER_TILE = _E // _NW
_N_CHUNKS = _E_PER_TILE // _CHUNK
_RPT = 3128
_RPT_LAST = _N - 15 * _RPT

_f32 = jnp.float32


_NBUF = 3


def _zero_accum(z_hbm, sh, acc0, last, zrows):
  pltpu.sync_copy(z_hbm, zrows)
  for j in range(3):
    pltpu.sync_copy(zrows, sh.at[pl.ds(acc0 + j * _CHUNK, _CHUNK)])

  @pl.when(jnp.logical_not(last))
  def _():
    pltpu.sync_copy(zrows.at[pl.ds(0, _RPT - 3 * _CHUNK)],
                    sh.at[pl.ds(acc0 + 3 * _CHUNK, _RPT - 3 * _CHUNK)])

  @pl.when(last)
  def _():
    pltpu.sync_copy(zrows.at[pl.ds(0, _RPT_LAST - 3 * _CHUNK)],
                    sh.at[pl.ds(acc0 + 3 * _CHUNK, _RPT_LAST - 3 * _CHUNK)])


def _write_out(sh, out, cid, acc0, last):
  @pl.when(jnp.logical_not(last))
  def _():
    pltpu.sync_copy(sh.at[pl.ds(acc0, _RPT)], out.at[cid, pl.ds(acc0, _RPT)])

  @pl.when(last)
  def _():
    pltpu.sync_copy(sh.at[pl.ds(acc0, _RPT_LAST)],
                    out.at[cid, pl.ds(acc0, _RPT_LAST)])


def _make_edge_agg():
  mesh = plsc.VectorSubcoreMesh(core_axis_name="c", subcore_axis_name="s")
  out_type = [jax.ShapeDtypeStruct((_NC, _N, _MSG), _f32)]
  scratch = [
      pltpu.VMEM((_E_PER_TILE,), jnp.int32),
      pltpu.VMEM((_NBUF, _CHUNK), jnp.int32),
      pltpu.VMEM((_NBUF, _CHUNK, _MSG), _f32),
      pltpu.VMEM_SHARED((_N, _MSG), _f32),
  ] + [pltpu.SemaphoreType.DMA] * (3 * _NBUF)

  @functools.partial(
      pl.kernel, mesh=mesh, out_type=out_type, scratch_types=scratch,
      compiler_params=pltpu.CompilerParams(use_tc_tiling_on_sc=False))
  def edge_agg(a_hbm, src_hbm, dst_hbm, z_hbm, s_out, src_v, dst_v, rows_v,
               s_sh, *sems):
    sem_i, sem_g, sem_s = sems[:_NBUF], sems[_NBUF:2 * _NBUF], sems[2 * _NBUF:]
    cid = lax.axis_index("c")
    sid = lax.axis_index("s")
    wid = cid * _NS + sid
    acc0 = sid * _RPT
    last = sid == _NS - 1
    base = wid * _E_PER_TILE

    _zero_accum(z_hbm, s_sh, acc0, last, rows_v.at[0])
    pltpu.sync_copy(src_hbm.at[pl.ds(base, _E_PER_TILE)], src_v)
    plsc.subcore_barrier()

    h_i = [None] * _NBUF
    h_g = [None] * _NBUF
    h_s = [None] * _NBUF
    for k in range(_NBUF):
      h_i[k] = pltpu.async_copy(
          dst_hbm.at[pl.ds(base + k * _CHUNK, _CHUNK)], dst_v.at[k], sem_i[k])
    for k in range(_N_CHUNKS + 1):
      b = k % _NBUF
      if k < _N_CHUNKS:
        if k >= _NBUF:
          h_s[b].wait()
          h_i[b] = pltpu.async_copy(
              dst_hbm.at[pl.ds(base + k * _CHUNK, _CHUNK)], dst_v.at[b],
              sem_i[b])
        h_g[b] = pltpu.async_copy(
            a_hbm.at[src_v.at[pl.ds(k * _CHUNK, _CHUNK)]], rows_v.at[b],
            sem_g[b])
      if k >= 1:
        bp = (k - 1) % _NBUF
        h_g[bp].wait()
        h_i[bp].wait()
        h_s[bp] = pltpu.async_copy(rows_v.at[bp], s_sh.at[dst_v.at[bp]],
                                   sem_s[bp], add=True)
    for k in range(_N_CHUNKS - _NBUF, _N_CHUNKS):
      h_s[k % _NBUF].wait()
    plsc.subcore_barrier()

    _write_out(s_sh, s_out, cid, acc0, last)

  return edge_agg


def _make_edge_count():
  mesh = plsc.VectorSubcoreMesh(core_axis_name="c", subcore_axis_name="s")
  out_type = [jax.ShapeDtypeStruct((_NC, _N, _MSG), _f32)]
  scratch = [
      pltpu.VMEM((_NBUF, _CHUNK), jnp.int32),
      pltpu.VMEM((_CHUNK, _MSG), _f32),
      pltpu.VMEM((_CHUNK, _MSG), _f32),
      pltpu.VMEM_SHARED((_N, _MSG), _f32),
  ] + [pltpu.SemaphoreType.DMA] * (2 * _NBUF)

  @functools.partial(
      pl.kernel, mesh=mesh, out_type=out_type, scratch_types=scratch,
      compiler_params=pltpu.CompilerParams(use_tc_tiling_on_sc=False))
  def edge_count(dst_hbm, z_hbm, o_hbm, c_out, dst_v, ones_v, zrows, c_sh,
                 *sems):
    sem_i, sem_s = sems[:_NBUF], sems[_NBUF:]
    cid = lax.axis_index("c")
    sid = lax.axis_index("s")
    wid = cid * _NS + sid
    acc0 = sid * _RPT
    last = sid == _NS - 1
    base = wid * _E_PER_TILE

    _zero_accum(z_hbm, c_sh, acc0, last, zrows)
    pltpu.sync_copy(o_hbm, ones_v)
    plsc.subcore_barrier()

    h_i = [None] * _NBUF
    h_s = [None] * _NBUF
    for k in range(_NBUF):
      h_i[k] = pltpu.async_copy(
          dst_hbm.at[pl.ds(base + k * _CHUNK, _CHUNK)], dst_v.at[k], sem_i[k])
    for k in range(_N_CHUNKS):
      b = k % _NBUF
      if k >= _NBUF:
        h_s[b].wait()
        h_i[b] = pltpu.async_copy(
            dst_hbm.at[pl.ds(base + k * _CHUNK, _CHUNK)], dst_v.at[b],
            sem_i[b])
      h_i[b].wait()
      h_s[b] = pltpu.async_copy(ones_v, c_sh.at[dst_v.at[b]], sem_s[b],
                                add=True)
    for k in range(_N_CHUNKS - _NBUF, _N_CHUNKS):
      h_s[k % _NBUF].wait()
    plsc.subcore_barrier()

    _write_out(c_sh, c_out, cid, acc0, last)

  return edge_count


_edge_agg = _make_edge_agg()
_edge_count = _make_edge_count()


_BN = 2000
_GRID = _N // _BN


def _softplus(x):
  return jnp.maximum(x, 0.0) + jnp.log1p(jnp.exp(-jnp.abs(x)))


def _dot(a, b):
  return jnp.dot(a, b, preferred_element_type=_f32,
                 precision=lax.Precision.HIGHEST)


def _full(shape):
  return pl.BlockSpec(shape, lambda i: (0, 0))


def _rows(width):
  return pl.BlockSpec((_BN, width), lambda i: (i, 0))


def _tc_pre(x, wms_t, wmd_t, ws_t, bm, bs):
  din = x.shape[1]

  def body(x_r, wms_r, wmd_r, ws_r, bm_r, bs_r, a_r, bp_r, xs_r):
    xv = x_r[...]
    a_r[...] = _dot(xv, wms_r[...])
    bp_r[...] = _dot(xv, wmd_r[...]) + bm_r[...]
    xs_r[...] = _dot(xv, ws_r[...]) + bs_r[...]

  return pl.pallas_call(
      body,
      grid=(_GRID,),
      in_specs=[_rows(din), _full(wms_t.shape), _full(wmd_t.shape),
                _full(ws_t.shape), _full(bm.shape), _full(bs.shape)],
      out_specs=[_rows(_MSG), _rows(_MSG), _rows(ws_t.shape[1])],
      out_shape=[jax.ShapeDtypeStruct((_N, _MSG), _f32),
                 jax.ShapeDtypeStruct((_N, _MSG), _f32),
                 jax.ShapeDtypeStruct((_N, ws_t.shape[1]), _f32)],
  )(x, wms_t, wmd_t, ws_t, bm, bs)


def _agg_from_parts(s0, s1, c0, c1, bp):
  cnt = c0 + c1
  ssum = s0 + s1
  return jnp.where(cnt > 0.5, ssum / jnp.maximum(cnt, 1.0) + bp, 0.0)


def _tc_mid(sp, cp, bp, xs, wu_t, bu, wms_t, wmd_t, ws_t, bm_n, bs_n):
  dnext = ws_t.shape[1]

  def body(s0_r, s1_r, c0_r, c1_r, bp_r, xs_r, wu_r, bu_r, wms_r, wmd_r,
           ws_r, bmn_r, bsn_r, a_r, bpn_r, xsn_r):
    agg = _agg_from_parts(s0_r[...], s1_r[...], c0_r[...], c1_r[...],
                          bp_r[...])
    h = _softplus(xs_r[...] + _dot(agg, wu_r[...]) + bu_r[...])
    a_r[...] = _dot(h, wms_r[...])
    bpn_r[...] = _dot(h, wmd_r[...]) + bmn_r[...]
    xsn_r[...] = _dot(h, ws_r[...]) + bsn_r[...]

  return pl.pallas_call(
      body,
      grid=(_GRID,),
      in_specs=[_rows(_MSG)] * 4 + [_rows(_MSG), _rows(xs.shape[1]),
                _full(wu_t.shape), _full(bu.shape), _full(wms_t.shape),
                _full(wmd_t.shape), _full(ws_t.shape), _full(bm_n.shape),
                _full(bs_n.shape)],
      out_specs=[_rows(_MSG), _rows(_MSG), _rows(dnext)],
      out_shape=[jax.ShapeDtypeStruct((_N, _MSG), _f32),
                 jax.ShapeDtypeStruct((_N, _MSG), _f32),
                 jax.ShapeDtypeStruct((_N, dnext), _f32)],
  )(sp[0], sp[1], cp[0], cp[1], bp, xs, wu_t, bu, wms_t, wmd_t, ws_t,
    bm_n, bs_n)


def _tc_final(sp, cp, bp, xs, wu_t, bu):
  dout = wu_t.shape[1]

  def body(s0_r, s1_r, c0_r, c1_r, bp_r, xs_r, wu_r, bu_r, o_r):
    agg = _agg_from_parts(s0_r[...], s1_r[...], c0_r[...], c1_r[...],
                          bp_r[...])
    o_r[...] = xs_r[...] + _dot(agg, wu_r[...]) + bu_r[...]

  return pl.pallas_call(
      body,
      grid=(_GRID,),
      in_specs=[_rows(_MSG)] * 4 + [_rows(_MSG), _rows(xs.shape[1]),
                _full(wu_t.shape), _full(bu.shape)],
      out_specs=_rows(dout),
      out_shape=jax.ShapeDtypeStruct((_N, dout), _f32),
  )(sp[0], sp[1], cp[0], cp[1], bp, xs, wu_t, bu)


def kernel(t, x_int_t, boundary_values, edge_index_int, edge_index_bound,
           timestamps, Wm1, bm1, Ws1, bs1, Wu1, bu1, Wm2, bm2, Ws2, bs2,
           Wu2, bu2, Wm3, bm3, Ws3, bs3, Wu3, bu3, Wm4, bm4, Ws4, bs4,
           Wu4, bu4):
  x = x_int_t[0]
  src = edge_index_int[0]
  dst = edge_index_int[1]

  layers = [(Wm1, bm1, Ws1, bs1, Wu1, bu1), (Wm2, bm2, Ws2, bs2, Wu2, bu2),
            (Wm3, bm3, Ws3, bs3, Wu3, bu3), (Wm4, bm4, Ws4, bs4, Wu4, bu4)]

  def split(l):
    Wm, bm, Ws, bs, Wu, bu = layers[l]
    din = Wm.shape[1] // 2
    return (Wm[:, :din].T, Wm[:, din:].T, Ws.T, Wu.T,
            bm.reshape(1, -1), bs.reshape(1, -1), bu.reshape(1, -1))

  wms, wmd, wst, wut, bm_, bs_, bu_ = zip(*[split(l) for l in range(4)])

  z16 = jnp.zeros((_CHUNK, _MSG), _f32)
  o16 = jnp.ones((_CHUNK, _MSG), _f32)
  cp = _edge_count(dst, z16, o16)[0]
  a, bp, xs = _tc_pre(x, wms[0], wmd[0], wst[0], bm_[0], bs_[0])
  sp = _edge_agg(a, src, dst, z16)[0]
  for l in (1, 2, 3):
    a, bp, xs = _tc_mid(sp, cp, bp, xs, wut[l - 1], bu_[l - 1], wms[l],
                        wmd[l], wst[l], bm_[l], bs_[l])
    sp = _edge_agg(a, src, dst, z16)[0]
  x_out = _tc_final(sp, cp, bp, xs, wut[3], bu_[3])

  return (x_out[None], jnp.zeros_like(boundary_values),
          jnp.zeros_like(edge_index_int), jnp.zeros_like(edge_index_bound),
          jnp.zeros_like(timestamps))
